# baseline (speedup 1.0000x reference)
import functools
import jax
import jax.numpy as jnp
from jax import lax
from jax.experimental import pallas as pl
from jax.experimental.pallas import tpu as pltpu
from jax.experimental.pallas import tpu_sc as plsc

N = 10000
E = 160000
D = 256
DH = 128

EK = 80
TROWS = 125
ACH = 100
ACN = 100

_mesh = functools.partial(
    plsc.VectorSubcoreMesh, core_axis_name="c", subcore_axis_name="s"
)


def _deg_kernel(dst3d_hbm, deg0_out, deg1_out, dstv, onesv, zbuf, acc):
    c = lax.axis_index("c")
    s = lax.axis_index("s")

    pltpu.sync_copy(dst3d_hbm.at[s], dstv)

    def fill(i, _):
        onesv[pl.ds(i * 16, 16)] = jnp.ones((16,), jnp.float32)
        return 0
    lax.fori_loop(0, EK // 16, fill, 0)

    def zfill(i, _):
        zbuf[pl.ds(i * 16, 16)] = jnp.zeros((16,), jnp.float32)
        return 0
    lax.fori_loop(0, 40, zfill, 0)

    pltpu.sync_copy(zbuf, acc.at[pl.ds(s * 640, 640)])

    plsc.subcore_barrier()

    def body(j, _):
        pltpu.sync_copy(onesv, acc.at[dstv.at[j]], add=True)
        return 0

    @pl.when(c == 0)
    def _():
        lax.fori_loop(0, 64, body, 0)

    @pl.when(c == 1)
    def _():
        lax.fori_loop(64, TROWS, body, 0)

    plsc.subcore_barrier()

    @pl.when(c == 0)
    def _():
        pltpu.sync_copy(acc.at[pl.ds(s * 640, 640)],
                        deg0_out.at[pl.ds(s * 640, 640)])

    @pl.when(c == 1)
    def _():
        pltpu.sync_copy(acc.at[pl.ds(s * 640, 640)],
                        deg1_out.at[pl.ds(s * 640, 640)])


NPAD = 10240

_deg_call = pl.kernel(
    _deg_kernel,
    out_type=[
        jax.ShapeDtypeStruct((NPAD,), jnp.float32),
        jax.ShapeDtypeStruct((NPAD,), jnp.float32),
    ],
    mesh=_mesh(),
    scratch_types=[
        pltpu.VMEM((TROWS, EK), jnp.int32),
        pltpu.VMEM((EK,), jnp.float32),
        pltpu.VMEM((640,), jnp.float32),
        pltpu.VMEM_SHARED((NPAD,), jnp.float32),
    ],
)


def _agg_kernel(g_lo, g_hi, eidx, out_lo, out_hi,
                ib0, ib1, ib2, ib3, ib4, ib5, buf0, buf1, buf2, acc,
                is0, is1, is2, is3, is4, is5, gs0, gs1, gs2,
                ss0, ss1, ss2):
    c = lax.axis_index("c")
    s = lax.axis_index("s")

    ibufs = (ib0, ib1, ib2, ib3, ib4, ib5)
    isems = (is0, is1, is2, is3, is4, is5)
    gbufs = (buf0, buf1, buf2)
    gsems = (gs0, gs1, gs2)
    ssems = (ss0, ss1, ss2)
    ebase = s * ACN

    roff = s * 640
    rlen0, rlen15 = 640, 400

    def run(g_half, out_half):
        for b in range(5):
            pltpu.async_copy(eidx.at[ebase + b], ibufs[b], isems[b])

        @pl.when(s < 15)
        def _():
            pltpu.sync_copy(g_half.at[pl.ds(roff, rlen0)],
                            acc.at[pl.ds(roff, rlen0)])

        @pl.when(s == 15)
        def _():
            pltpu.sync_copy(g_half.at[pl.ds(9600, rlen15)],
                            acc.at[pl.ds(9600, rlen15)])

        for b in range(2):
            pltpu.make_async_copy(eidx.at[ebase + b], ibufs[b],
                                  isems[b]).wait()

        plsc.subcore_barrier()

        for b in range(2):
            pltpu.async_copy(g_half.at[ibufs[b].at[0]], gbufs[b], gsems[b])

        def step(j, b3, i6, last3, idx5):
            pltpu.make_async_copy(g_half.at[ibufs[i6].at[0]], gbufs[b3],
                                  gsems[b3]).wait()
            pltpu.async_copy(gbufs[b3], acc.at[ibufs[i6].at[1]],
                             ssems[b3], add=True)

            def wait_prev():
                pltpu.make_async_copy(gbufs[last3],
                                      acc.at[ibufs[i6].at[1]],
                                      ssems[last3]).wait()

            if isinstance(j, int):
                if j > 0:
                    wait_prev()
            else:
                pl.when(j > 0)(wait_prev)

            if idx5:
                pltpu.async_copy(eidx.at[ebase + j + 5], ibufs[(i6 + 5) % 6],
                                 isems[(i6 + 5) % 6])
            if idx5 or isinstance(j, int):
                if not isinstance(j, int) or j + 2 < ACN:
                    pltpu.make_async_copy(eidx.at[ebase + j + 2],
                                          ibufs[(i6 + 2) % 6],
                                          isems[(i6 + 2) % 6]).wait()
                    pltpu.async_copy(g_half.at[ibufs[(i6 + 2) % 6].at[0]],
                                     gbufs[(b3 + 2) % 3],
                                     gsems[(b3 + 2) % 3])

        def block(jb, _):
            for k in range(6):
                j = 6 * jb + k
                step(j, k % 3, k, (k + 2) % 3, True)
            return 0

        nblk = (ACN - 5) // 6
        lax.fori_loop(0, nblk, block, 0)

        for j in range(6 * nblk, ACN):
            step(j, j % 3, j % 6, (j + 2) % 3, j + 5 < ACN)

        pltpu.make_async_copy(gbufs[(ACN - 1) % 3],
                              acc.at[ibufs[(ACN - 1) % 6].at[1]],
                              ssems[(ACN - 1) % 3]).wait()

        plsc.subcore_barrier()

        @pl.when(s < 15)
        def _():
            pltpu.sync_copy(acc.at[pl.ds(roff, rlen0)],
                            out_half.at[pl.ds(roff, rlen0)])

        @pl.when(s == 15)
        def _():
            pltpu.sync_copy(acc.at[pl.ds(9600, rlen15)],
                            out_half.at[pl.ds(9600, rlen15)])

    @pl.when(c == 0)
    def _():
        run(g_lo, out_lo)

    @pl.when(c == 1)
    def _():
        run(g_hi, out_hi)


_agg_call = pl.kernel(
    _agg_kernel,
    out_type=[
        jax.ShapeDtypeStruct((N, DH), jnp.float32),
        jax.ShapeDtypeStruct((N, DH), jnp.float32),
    ],
    mesh=_mesh(),
    scratch_types=(
        [pltpu.VMEM((2, ACH), jnp.int32)] * 6
        + [pltpu.VMEM((ACH, DH), jnp.float32)] * 3
        + [pltpu.VMEM_SHARED((N, DH), jnp.float32)]
        + [pltpu.SemaphoreType.DMA] * 12
    ),
)


ROWB = 1000


def _dinv(deg0_ref, deg1_ref):
    deg = deg0_ref[...] + deg1_ref[...] + 1.0
    return lax.rsqrt(jnp.maximum(deg, 1.0))


def _mm(h, w_ref):
    return lax.dot_general(
        h, w_ref[...], (((1,), (0,)), ((), ())),
        preferred_element_type=jnp.float32,
        precision=lax.Precision.HIGHEST,
    )


def _tc_first_kernel(x_ref, deg0_ref, deg1_ref, w_ref, glo_ref, ghi_ref):
    dinv = _dinv(deg0_ref, deg1_ref)
    g = _mm(x_ref[...], w_ref) * dinv
    glo_ref[...] = g[:, :DH]
    ghi_ref[...] = g[:, DH:]


def _tc_mid_kernel(alo_ref, ahi_ref, deg0_ref, deg1_ref, w_ref,
                   glo_ref, ghi_ref):
    dinv = _dinv(deg0_ref, deg1_ref)
    h = jnp.concatenate([alo_ref[...], ahi_ref[...]], axis=1) * dinv
    h = jnp.maximum(h, 0.0)
    g = _mm(h, w_ref) * dinv
    glo_ref[...] = g[:, :DH]
    ghi_ref[...] = g[:, DH:]


def _tc_last_kernel(alo_ref, ahi_ref, deg0_ref, deg1_ref, out_ref):
    dinv = _dinv(deg0_ref, deg1_ref)
    h = jnp.concatenate([alo_ref[...], ahi_ref[...]], axis=1) * dinv
    h = jnp.maximum(h, 0.0)
    nrm = jnp.sqrt(jnp.sum(h * h, axis=1, keepdims=True))
    out_ref[...] = h / jnp.maximum(nrm, 1e-12)


def _row_spec(cols):
    return pl.BlockSpec((ROWB, cols), lambda i: (i, 0))


_W_SPEC = pl.BlockSpec((D, D), lambda i: (0, 0))
_GRID = (N // ROWB,)
_HALF_OUT = [
    jax.ShapeDtypeStruct((N, DH), jnp.float32),
    jax.ShapeDtypeStruct((N, DH), jnp.float32),
]

_tc_first = pl.pallas_call(
    _tc_first_kernel,
    grid=_GRID,
    in_specs=[_row_spec(D), _row_spec(1), _row_spec(1), _W_SPEC],
    out_specs=[_row_spec(DH), _row_spec(DH)],
    out_shape=_HALF_OUT,
)

_tc_mid = pl.pallas_call(
    _tc_mid_kernel,
    grid=_GRID,
    in_specs=[_row_spec(DH), _row_spec(DH), _row_spec(1), _row_spec(1),
              _W_SPEC],
    out_specs=[_row_spec(DH), _row_spec(DH)],
    out_shape=_HALF_OUT,
)

_tc_last = pl.pallas_call(
    _tc_last_kernel,
    grid=_GRID,
    in_specs=[_row_spec(DH), _row_spec(DH), _row_spec(1), _row_spec(1)],
    out_specs=_row_spec(D),
    out_shape=jax.ShapeDtypeStruct((N, D), jnp.float32),
)


@jax.jit
def kernel(x, edge_index, W0, W1, W2):
    eidx = jnp.stack(
        [edge_index[0].reshape(16 * ACN, ACH),
         edge_index[1].reshape(16 * ACN, ACH)], axis=1)
    dst_deg = edge_index[1].reshape(16, TROWS, EK)

    deg0, deg1 = _deg_call(dst_deg)
    d0 = deg0[:N].reshape(N, 1)
    d1 = deg1[:N].reshape(N, 1)

    glo, ghi = _tc_first(x, d0, d1, W0)
    alo, ahi = _agg_call(glo, ghi, eidx)
    glo, ghi = _tc_mid(alo, ahi, d0, d1, W1)
    alo, ahi = _agg_call(glo, ghi, eidx)
    glo, ghi = _tc_mid(alo, ahi, d0, d1, W2)
    alo, ahi = _agg_call(glo, ghi, eidx)
    return _tc_last(alo, ahi, d0, d1)


# speedup vs baseline: 1.0062x; 1.0062x over previous
import functools
import jax
import jax.numpy as jnp
from jax import lax
from jax.experimental import pallas as pl
from jax.experimental.pallas import tpu as pltpu
from jax.experimental.pallas import tpu_sc as plsc

N = 10000
E = 160000
D = 256
DH = 128

EK = 80
TROWS = 125
ACH = 80
ACN = 125

_mesh = functools.partial(
    plsc.VectorSubcoreMesh, core_axis_name="c", subcore_axis_name="s"
)


def _deg_kernel(dst3d_hbm, deg_out, dstv, onesv, zbuf, acc):
    c = lax.axis_index("c")
    s = lax.axis_index("s")

    @pl.when(c == 0)
    def _():
        pltpu.sync_copy(dst3d_hbm.at[s], dstv)

        def fill(i, _):
            onesv[pl.ds(i * 16, 16)] = jnp.ones((16,), jnp.float32)
            return 0
        lax.fori_loop(0, EK // 16, fill, 0)

        def zfill(i, _):
            zbuf[pl.ds(i * 16, 16)] = jnp.zeros((16,), jnp.float32)
            return 0
        lax.fori_loop(0, 40, zfill, 0)

        pltpu.sync_copy(zbuf, acc.at[pl.ds(s * 640, 640)])

        plsc.subcore_barrier()

        def body(j, _):
            pltpu.sync_copy(onesv, acc.at[dstv.at[j]], add=True)
            return 0
        lax.fori_loop(0, TROWS, body, 0)

        plsc.subcore_barrier()

        pltpu.sync_copy(acc.at[pl.ds(s * 640, 640)],
                        deg_out.at[pl.ds(s * 640, 640)])


NPAD = 10240

_deg_call = pl.kernel(
    _deg_kernel,
    out_type=jax.ShapeDtypeStruct((NPAD,), jnp.float32),
    mesh=_mesh(),
    scratch_types=[
        pltpu.VMEM((TROWS, EK), jnp.int32),
        pltpu.VMEM((EK,), jnp.float32),
        pltpu.VMEM((640,), jnp.float32),
        pltpu.VMEM_SHARED((NPAD,), jnp.float32),
    ],
)


def _agg_kernel(g_lo, g_hi, eidx, out_lo, out_hi,
                ib0, ib1, ib2, ib3, ib4, ib5, buf0, buf1, buf2, acc,
                is0, is1, is2, is3, is4, is5, gs0, gs1, gs2,
                ss0, ss1, ss2):
    c = lax.axis_index("c")
    s = lax.axis_index("s")

    ibufs = (ib0, ib1, ib2, ib3, ib4, ib5)
    isems = (is0, is1, is2, is3, is4, is5)
    gbufs = (buf0, buf1, buf2)
    gsems = (gs0, gs1, gs2)
    ssems = (ss0, ss1, ss2)
    ebase = s * ACN

    roff = s * 640
    rlen0, rlen15 = 640, 400

    def run(g_half, out_half):
        @pl.when(s < 15)
        def _():
            pltpu.sync_copy(g_half.at[pl.ds(roff, rlen0)],
                            acc.at[pl.ds(roff, rlen0)])

        @pl.when(s == 15)
        def _():
            pltpu.sync_copy(g_half.at[pl.ds(9600, rlen15)],
                            acc.at[pl.ds(9600, rlen15)])

        for b in range(5):
            pltpu.async_copy(eidx.at[ebase + b], ibufs[b], isems[b])
        for b in range(2):
            pltpu.make_async_copy(eidx.at[ebase + b], ibufs[b],
                                  isems[b]).wait()

        plsc.subcore_barrier()

        for b in range(2):
            pltpu.async_copy(g_half.at[ibufs[b].at[0]], gbufs[b], gsems[b])

        def step(j, b3, i6, last3, idx5):
            pltpu.make_async_copy(g_half.at[ibufs[i6].at[0]], gbufs[b3],
                                  gsems[b3]).wait()
            pltpu.async_copy(gbufs[b3], acc.at[ibufs[i6].at[1]],
                             ssems[b3], add=True)

            def wait_prev():
                pltpu.make_async_copy(gbufs[last3],
                                      acc.at[ibufs[i6].at[1]],
                                      ssems[last3]).wait()

            if isinstance(j, int):
                if j > 0:
                    wait_prev()
            else:
                pl.when(j > 0)(wait_prev)

            if idx5:
                pltpu.async_copy(eidx.at[ebase + j + 5], ibufs[(i6 + 5) % 6],
                                 isems[(i6 + 5) % 6])
            if idx5 or isinstance(j, int):
                if not isinstance(j, int) or j + 2 < ACN:
                    pltpu.make_async_copy(eidx.at[ebase + j + 2],
                                          ibufs[(i6 + 2) % 6],
                                          isems[(i6 + 2) % 6]).wait()
                    pltpu.async_copy(g_half.at[ibufs[(i6 + 2) % 6].at[0]],
                                     gbufs[(b3 + 2) % 3],
                                     gsems[(b3 + 2) % 3])

        def block(jb, _):
            for k in range(6):
                j = 6 * jb + k
                step(j, k % 3, k, (k + 2) % 3, True)
            return 0

        nblk = (ACN - 5) // 6
        lax.fori_loop(0, nblk, block, 0)

        for j in range(6 * nblk, ACN):
            step(j, j % 3, j % 6, (j + 2) % 3, j + 5 < ACN)

        pltpu.make_async_copy(gbufs[(ACN - 1) % 3],
                              acc.at[ibufs[(ACN - 1) % 6].at[1]],
                              ssems[(ACN - 1) % 3]).wait()

        plsc.subcore_barrier()

        @pl.when(s < 15)
        def _():
            pltpu.sync_copy(acc.at[pl.ds(roff, rlen0)],
                            out_half.at[pl.ds(roff, rlen0)])

        @pl.when(s == 15)
        def _():
            pltpu.sync_copy(acc.at[pl.ds(9600, rlen15)],
                            out_half.at[pl.ds(9600, rlen15)])

    @pl.when(c == 0)
    def _():
        run(g_lo, out_lo)

    @pl.when(c == 1)
    def _():
        run(g_hi, out_hi)


_agg_call = pl.kernel(
    _agg_kernel,
    out_type=[
        jax.ShapeDtypeStruct((N, DH), jnp.float32),
        jax.ShapeDtypeStruct((N, DH), jnp.float32),
    ],
    mesh=_mesh(),
    scratch_types=(
        [pltpu.VMEM((2, ACH), jnp.int32)] * 6
        + [pltpu.VMEM((ACH, DH), jnp.float32)] * 3
        + [pltpu.VMEM_SHARED((N, DH), jnp.float32)]
        + [pltpu.SemaphoreType.DMA] * 12
    ),
)


ROWB = 1000


def _dinv(deg_ref):
    deg = deg_ref[...] + 1.0
    return lax.rsqrt(jnp.maximum(deg, 1.0))


def _mm(h, w_ref):
    return lax.dot_general(
        h, w_ref[...], (((1,), (0,)), ((), ())),
        preferred_element_type=jnp.float32,
        precision=lax.Precision.HIGHEST,
    )


def _tc_first_kernel(x_ref, deg_ref, w_ref, glo_ref, ghi_ref):
    dinv = _dinv(deg_ref)
    g = _mm(x_ref[...], w_ref) * dinv
    glo_ref[...] = g[:, :DH]
    ghi_ref[...] = g[:, DH:]


def _tc_mid_kernel(alo_ref, ahi_ref, deg_ref, w_ref, glo_ref, ghi_ref):
    dinv = _dinv(deg_ref)
    h = jnp.concatenate([alo_ref[...], ahi_ref[...]], axis=1) * dinv
    h = jnp.maximum(h, 0.0)
    g = _mm(h, w_ref) * dinv
    glo_ref[...] = g[:, :DH]
    ghi_ref[...] = g[:, DH:]


def _tc_last_kernel(alo_ref, ahi_ref, deg_ref, out_ref):
    dinv = _dinv(deg_ref)
    h = jnp.concatenate([alo_ref[...], ahi_ref[...]], axis=1) * dinv
    h = jnp.maximum(h, 0.0)
    nrm = jnp.sqrt(jnp.sum(h * h, axis=1, keepdims=True))
    out_ref[...] = h / jnp.maximum(nrm, 1e-12)


def _row_spec(cols):
    return pl.BlockSpec((ROWB, cols), lambda i: (i, 0))


_W_SPEC = pl.BlockSpec((D, D), lambda i: (0, 0))
_GRID = (N // ROWB,)
_HALF_OUT = [
    jax.ShapeDtypeStruct((N, DH), jnp.float32),
    jax.ShapeDtypeStruct((N, DH), jnp.float32),
]

_tc_first = pl.pallas_call(
    _tc_first_kernel,
    grid=_GRID,
    in_specs=[_row_spec(D), _row_spec(1), _W_SPEC],
    out_specs=[_row_spec(DH), _row_spec(DH)],
    out_shape=_HALF_OUT,
)

_tc_mid = pl.pallas_call(
    _tc_mid_kernel,
    grid=_GRID,
    in_specs=[_row_spec(DH), _row_spec(DH), _row_spec(1), _W_SPEC],
    out_specs=[_row_spec(DH), _row_spec(DH)],
    out_shape=_HALF_OUT,
)

_tc_last = pl.pallas_call(
    _tc_last_kernel,
    grid=_GRID,
    in_specs=[_row_spec(DH), _row_spec(DH), _row_spec(1)],
    out_specs=_row_spec(D),
    out_shape=jax.ShapeDtypeStruct((N, D), jnp.float32),
)


@jax.jit
def kernel(x, edge_index, W0, W1, W2):
    eidx = jnp.stack(
        [edge_index[0].reshape(16 * ACN, ACH),
         edge_index[1].reshape(16 * ACN, ACH)], axis=1)
    dst_deg = edge_index[1].reshape(16, TROWS, EK)

    deg = _deg_call(dst_deg)
    deg2 = deg[:N].reshape(N, 1)

    glo, ghi = _tc_first(x, deg2, W0)
    alo, ahi = _agg_call(glo, ghi, eidx)
    glo, ghi = _tc_mid(alo, ahi, deg2, W1)
    alo, ahi = _agg_call(glo, ghi, eidx)
    glo, ghi = _tc_mid(alo, ahi, deg2, W2)
    alo, ahi = _agg_call(glo, ghi, eidx)
    return _tc_last(alo, ahi, deg2)


# speedup vs baseline: 1.0251x; 1.0188x over previous
import functools
import jax
import jax.numpy as jnp
from jax import lax
from jax.experimental import pallas as pl
from jax.experimental.pallas import tpu as pltpu
from jax.experimental.pallas import tpu_sc as plsc

N = 10000
E = 160000
D = 256
DH = 128

EK = 80
TROWS = 125
ACH = 80
ACN = 125

_mesh = functools.partial(
    plsc.VectorSubcoreMesh, core_axis_name="c", subcore_axis_name="s"
)


def _deg_kernel(dst3d_hbm, deg_out, dstv, onesv, zbuf, acc, dsem):
    c = lax.axis_index("c")
    s = lax.axis_index("s")

    @pl.when(c == 0)
    def _():
        pltpu.sync_copy(dst3d_hbm.at[s], dstv)

        def fill(i, _):
            onesv[pl.ds(i * 16, 16)] = jnp.ones((16,), jnp.float32)
            return 0
        lax.fori_loop(0, EK // 16, fill, 0)

        def zfill(i, _):
            zbuf[pl.ds(i * 16, 16)] = jnp.zeros((16,), jnp.float32)
            return 0
        lax.fori_loop(0, 40, zfill, 0)

        pltpu.sync_copy(zbuf, acc.at[pl.ds(s * 640, 640)])

        plsc.subcore_barrier()

        def body(j, _):
            pltpu.async_copy(onesv, acc.at[dstv.at[j]], dsem, add=True)
            return 0
        lax.fori_loop(0, TROWS, body, 0)

        def drain(j, _):
            pltpu.make_async_copy(onesv, acc.at[dstv.at[j]], dsem).wait()
            return 0
        lax.fori_loop(0, TROWS, drain, 0)

        plsc.subcore_barrier()

        pltpu.sync_copy(acc.at[pl.ds(s * 640, 640)],
                        deg_out.at[pl.ds(s * 640, 640)])


NPAD = 10240

_deg_call = pl.kernel(
    _deg_kernel,
    out_type=jax.ShapeDtypeStruct((NPAD,), jnp.float32),
    mesh=_mesh(),
    scratch_types=[
        pltpu.VMEM((TROWS, EK), jnp.int32),
        pltpu.VMEM((EK,), jnp.float32),
        pltpu.VMEM((640,), jnp.float32),
        pltpu.VMEM_SHARED((NPAD,), jnp.float32),
        pltpu.SemaphoreType.DMA,
    ],
)


def _agg_kernel(g_lo, g_hi, eidx, out_lo, out_hi,
                ib0, ib1, ib2, ib3, ib4, ib5, buf0, buf1, buf2, acc,
                is0, is1, is2, is3, is4, is5, gs0, gs1, gs2,
                ss0, ss1, ss2):
    c = lax.axis_index("c")
    s = lax.axis_index("s")

    ibufs = (ib0, ib1, ib2, ib3, ib4, ib5)
    isems = (is0, is1, is2, is3, is4, is5)
    gbufs = (buf0, buf1, buf2)
    gsems = (gs0, gs1, gs2)
    ssems = (ss0, ss1, ss2)
    ebase = s * ACN

    roff = s * 640
    rlen0, rlen15 = 640, 400

    def run(g_half, out_half):
        @pl.when(s < 15)
        def _():
            pltpu.sync_copy(g_half.at[pl.ds(roff, rlen0)],
                            acc.at[pl.ds(roff, rlen0)])

        @pl.when(s == 15)
        def _():
            pltpu.sync_copy(g_half.at[pl.ds(9600, rlen15)],
                            acc.at[pl.ds(9600, rlen15)])

        for b in range(5):
            pltpu.async_copy(eidx.at[ebase + b], ibufs[b], isems[b])
        for b in range(2):
            pltpu.make_async_copy(eidx.at[ebase + b], ibufs[b],
                                  isems[b]).wait()

        plsc.subcore_barrier()

        for b in range(2):
            pltpu.async_copy(g_half.at[ibufs[b].at[0]], gbufs[b], gsems[b])

        def step(j, b3, i6, last3, idx5):
            pltpu.make_async_copy(g_half.at[ibufs[i6].at[0]], gbufs[b3],
                                  gsems[b3]).wait()
            pltpu.async_copy(gbufs[b3], acc.at[ibufs[i6].at[1]],
                             ssems[b3], add=True)

            def wait_prev():
                pltpu.make_async_copy(gbufs[last3],
                                      acc.at[ibufs[i6].at[1]],
                                      ssems[last3]).wait()

            if isinstance(j, int):
                if j > 0:
                    wait_prev()
            else:
                pl.when(j > 0)(wait_prev)

            if idx5:
                pltpu.async_copy(eidx.at[ebase + j + 5], ibufs[(i6 + 5) % 6],
                                 isems[(i6 + 5) % 6])
            if idx5 or isinstance(j, int):
                if not isinstance(j, int) or j + 2 < ACN:
                    pltpu.make_async_copy(eidx.at[ebase + j + 2],
                                          ibufs[(i6 + 2) % 6],
                                          isems[(i6 + 2) % 6]).wait()
                    pltpu.async_copy(g_half.at[ibufs[(i6 + 2) % 6].at[0]],
                                     gbufs[(b3 + 2) % 3],
                                     gsems[(b3 + 2) % 3])

        def block(jb, _):
            for k in range(6):
                j = 6 * jb + k
                step(j, k % 3, k, (k + 2) % 3, True)
            return 0

        nblk = (ACN - 5) // 6
        lax.fori_loop(0, nblk, block, 0)

        for j in range(6 * nblk, ACN):
            step(j, j % 3, j % 6, (j + 2) % 3, j + 5 < ACN)

        pltpu.make_async_copy(gbufs[(ACN - 1) % 3],
                              acc.at[ibufs[(ACN - 1) % 6].at[1]],
                              ssems[(ACN - 1) % 3]).wait()

        plsc.subcore_barrier()

        @pl.when(s < 15)
        def _():
            pltpu.sync_copy(acc.at[pl.ds(roff, rlen0)],
                            out_half.at[pl.ds(roff, rlen0)])

        @pl.when(s == 15)
        def _():
            pltpu.sync_copy(acc.at[pl.ds(9600, rlen15)],
                            out_half.at[pl.ds(9600, rlen15)])

    @pl.when(c == 0)
    def _():
        run(g_lo, out_lo)

    @pl.when(c == 1)
    def _():
        run(g_hi, out_hi)


_agg_call = pl.kernel(
    _agg_kernel,
    out_type=[
        jax.ShapeDtypeStruct((N, DH), jnp.float32),
        jax.ShapeDtypeStruct((N, DH), jnp.float32),
    ],
    mesh=_mesh(),
    scratch_types=(
        [pltpu.VMEM((2, ACH), jnp.int32)] * 6
        + [pltpu.VMEM((ACH, DH), jnp.float32)] * 3
        + [pltpu.VMEM_SHARED((N, DH), jnp.float32)]
        + [pltpu.SemaphoreType.DMA] * 12
    ),
)


ROWB = 1000


def _dinv(deg_ref):
    deg = deg_ref[...] + 1.0
    return lax.rsqrt(jnp.maximum(deg, 1.0))


def _mm(h, w_ref):
    return lax.dot_general(
        h, w_ref[...], (((1,), (0,)), ((), ())),
        preferred_element_type=jnp.float32,
        precision=lax.Precision.HIGHEST,
    )


def _tc_first_kernel(x_ref, deg_ref, w_ref, glo_ref, ghi_ref):
    dinv = _dinv(deg_ref)
    g = _mm(x_ref[...], w_ref) * dinv
    glo_ref[...] = g[:, :DH]
    ghi_ref[...] = g[:, DH:]


def _tc_mid_kernel(alo_ref, ahi_ref, deg_ref, w_ref, glo_ref, ghi_ref):
    dinv = _dinv(deg_ref)
    h = jnp.concatenate([alo_ref[...], ahi_ref[...]], axis=1) * dinv
    h = jnp.maximum(h, 0.0)
    g = _mm(h, w_ref) * dinv
    glo_ref[...] = g[:, :DH]
    ghi_ref[...] = g[:, DH:]


def _tc_last_kernel(alo_ref, ahi_ref, deg_ref, out_ref):
    dinv = _dinv(deg_ref)
    h = jnp.concatenate([alo_ref[...], ahi_ref[...]], axis=1) * dinv
    h = jnp.maximum(h, 0.0)
    nrm = jnp.sqrt(jnp.sum(h * h, axis=1, keepdims=True))
    out_ref[...] = h / jnp.maximum(nrm, 1e-12)


def _row_spec(cols):
    return pl.BlockSpec((ROWB, cols), lambda i: (i, 0))


_W_SPEC = pl.BlockSpec((D, D), lambda i: (0, 0))
_GRID = (N // ROWB,)
_HALF_OUT = [
    jax.ShapeDtypeStruct((N, DH), jnp.float32),
    jax.ShapeDtypeStruct((N, DH), jnp.float32),
]

_tc_first = pl.pallas_call(
    _tc_first_kernel,
    grid=_GRID,
    in_specs=[_row_spec(D), _row_spec(1), _W_SPEC],
    out_specs=[_row_spec(DH), _row_spec(DH)],
    out_shape=_HALF_OUT,
)

_tc_mid = pl.pallas_call(
    _tc_mid_kernel,
    grid=_GRID,
    in_specs=[_row_spec(DH), _row_spec(DH), _row_spec(1), _W_SPEC],
    out_specs=[_row_spec(DH), _row_spec(DH)],
    out_shape=_HALF_OUT,
)

_tc_last = pl.pallas_call(
    _tc_last_kernel,
    grid=_GRID,
    in_specs=[_row_spec(DH), _row_spec(DH), _row_spec(1)],
    out_specs=_row_spec(D),
    out_shape=jax.ShapeDtypeStruct((N, D), jnp.float32),
)


@jax.jit
def kernel(x, edge_index, W0, W1, W2):
    eidx = jnp.stack(
        [edge_index[0].reshape(16 * ACN, ACH),
         edge_index[1].reshape(16 * ACN, ACH)], axis=1)
    dst_deg = edge_index[1].reshape(16, TROWS, EK)

    deg = _deg_call(dst_deg)
    deg2 = deg[:N].reshape(N, 1)

    glo, ghi = _tc_first(x, deg2, W0)
    alo, ahi = _agg_call(glo, ghi, eidx)
    glo, ghi = _tc_mid(alo, ahi, deg2, W1)
    alo, ahi = _agg_call(glo, ghi, eidx)
    glo, ghi = _tc_mid(alo, ahi, deg2, W2)
    alo, ahi = _agg_call(glo, ghi, eidx)
    return _tc_last(alo, ahi, deg2)
